# speedup vs baseline: 4.4404x; 4.4404x over previous
import jax
import jax.numpy as jnp
from jax import lax
from jax.experimental import pallas as pl


_K = 9
_S = 1 << _K
_V = 10
_N = 18


def _tc_body(c2_ref, m11_ref, m22_ref, out_ref):
    rows = lax.broadcasted_iota(jnp.int32, (_S, 128), 0)
    cols = lax.broadcasted_iota(jnp.int32, (_S, 128), 1)
    bits = jnp.where(cols < _K, (rows >> jnp.minimum(cols, _K)) & 1, 0)
    bits = bits.astype(jnp.float32)

    hp = jax.lax.Precision.HIGHEST
    dn_row = (((1,), (0,)), ((), ()))
    dn_tt = (((1,), (1,)), ((), ()))

    h11 = lax.dot_general(bits, m11_ref[...], dn_row,
                          precision=hp, preferred_element_type=jnp.float32)
    ea = jnp.sum(h11 * bits, axis=1, keepdims=True)
    h22 = lax.dot_general(bits, m22_ref[...], dn_row,
                          precision=hp, preferred_element_type=jnp.float32)
    eb = jnp.sum(h22 * bits, axis=1, keepdims=True)
    g = lax.dot_general(bits, c2_ref[...], dn_row,
                        precision=hp, preferred_element_type=jnp.float32)

    gaug = jnp.where(cols == _K, 1.0, jnp.where(cols == _K + 1, ea, g))
    xaug = jnp.where(cols == _K, eb, jnp.where(cols == _K + 1, 1.0, bits))
    t = lax.dot_general(gaug, xaug, dn_tt,
                        precision=hp, preferred_element_type=jnp.float32)

    w = jnp.exp(t)
    r = jnp.sum(w, axis=1, keepdims=True)
    c = jnp.sum(w, axis=0, keepdims=True)
    z = jnp.sum(r)
    plo = lax.dot_general(r, bits, (((0,), (0,)), ((), ())),
                          precision=hp, preferred_element_type=jnp.float32)
    phi = lax.dot_general(c, bits, dn_row,
                          precision=hp, preferred_element_type=jnp.float32)
    out_ref[0:1, :] = plo / z
    out_ref[1:2, :] = phi / z


def kernel(matrix, beta):
    m = beta * matrix.astype(jnp.float32)
    zpad = jnp.zeros((128, 128), jnp.float32)
    c2 = zpad.at[:_K, :_K].set(2.0 * m[:_K, _K:])
    m11 = zpad.at[:_K, :_K].set(m[:_K, :_K])
    m22 = zpad.at[:_K, :_K].set(m[_K:, _K:])

    out = pl.pallas_call(
        _tc_body,
        out_shape=jax.ShapeDtypeStruct((8, 128), jnp.float32),
    )(c2, m11, m22)

    prob = jnp.concatenate([out[0, :_K], out[1, :_K]])
    return prob[None, :_V], prob[None, _V:_N]


# speedup vs baseline: 9.5529x; 2.1513x over previous
import jax
import jax.numpy as jnp
from jax import lax
from jax.experimental import pallas as pl
from jax.experimental.pallas import tpu as pltpu


_K = 9
_S = 1 << _K
_V = 10
_N = 18


def _tc_body(m_ref, beta_ref, vs_ref, hs_ref):
    beta = beta_ref[0]
    mm = m_ref[...] * beta
    mfull = jnp.concatenate(
        [jnp.concatenate([mm, jnp.zeros((_N, 128 - _N), jnp.float32)], axis=1),
         jnp.zeros((128 - _N, 128), jnp.float32)], axis=0)

    rows = lax.broadcasted_iota(jnp.int32, (_S, 128), 0)
    cols = lax.broadcasted_iota(jnp.int32, (_S, 128), 1)
    bl = jnp.where(cols < _K, (rows >> jnp.minimum(cols, _K)) & 1, 0)
    bl = bl.astype(jnp.float32)
    bh = jnp.where((cols >= _K) & (cols < _N),
                   (rows >> jnp.clip(cols - _K, 0, _K)) & 1, 0)
    bh = bh.astype(jnp.float32)

    hp = jax.lax.Precision.HIGHEST
    dn_row = (((1,), (0,)), ((), ()))
    dn_tt = (((1,), (1,)), ((), ()))

    hl = lax.dot_general(bl, mfull, dn_row,
                         precision=hp, preferred_element_type=jnp.float32)
    ea = jnp.sum(hl * bl, axis=1, keepdims=True)
    hh = lax.dot_general(bh, mfull, dn_row,
                         precision=hp, preferred_element_type=jnp.float32)
    eb = jnp.sum(hh * bh, axis=1, keepdims=True)

    gaug = jnp.where(cols == _N, 1.0, jnp.where(cols == _N + 1, ea, 2.0 * hl))
    xaug = jnp.where(cols == _N, eb, jnp.where(cols == _N + 1, 1.0, bh))
    t = lax.dot_general(gaug, xaug, dn_tt,
                        precision=hp, preferred_element_type=jnp.float32)

    w = jnp.exp(t)
    r = jnp.sum(w, axis=1, keepdims=True)
    c = jnp.sum(w, axis=0, keepdims=True)
    z = jnp.sum(r)
    plo = lax.dot_general(r, bl, (((0,), (0,)), ((), ())),
                          precision=hp, preferred_element_type=jnp.float32)
    phi = lax.dot_general(c, bh, dn_row,
                          precision=hp, preferred_element_type=jnp.float32)
    prob = (plo + phi) / z
    vs_ref[...] = prob[:, :_V]
    hs_ref[...] = prob[:, _V:_N]


def kernel(matrix, beta):
    return pl.pallas_call(
        _tc_body,
        in_specs=[
            pl.BlockSpec(memory_space=pltpu.VMEM),
            pl.BlockSpec(memory_space=pltpu.SMEM),
        ],
        out_shape=(jax.ShapeDtypeStruct((1, _V), jnp.float32),
                   jax.ShapeDtypeStruct((1, _N - _V), jnp.float32)),
    )(matrix.astype(jnp.float32), beta.reshape(1).astype(jnp.float32))
